# baseline (speedup 1.0000x reference)
import functools

import jax
import jax.numpy as jnp
from jax.experimental import pallas as pl
from jax.experimental.pallas import tpu as pltpu

N, D_IN = 16384, 768
H1, H2 = 1024, 256
NUM_CODES, CODE_DIM = 256, 256
COMMITMENT_COST = 0.25

TILE = 512


def _fused_body(x_ref, W1_ref, b1_ref, W2_ref, b2_ref, cb_ref, w2sum_ref,
                W3_ref, b3_ref, W4_ref, b4_ref, out_ref, loss_ref):
    x = x_ref[...]
    h = jnp.maximum(
        jnp.dot(x, W1_ref[...], preferred_element_type=jnp.float32) + b1_ref[...], 0.0)
    z = jnp.maximum(
        jnp.dot(h, W2_ref[...], preferred_element_type=jnp.float32) + b2_ref[...], 0.0)

    zc = jnp.dot(z, cb_ref[...], preferred_element_type=jnp.float32)
    z2 = jnp.sum(z * z, axis=1, keepdims=True)
    d2 = jnp.maximum(z2 + w2sum_ref[...] - 2.0 * zc, 0.0)
    idx = jnp.argmin(d2, axis=1)

    onehot = (jax.lax.broadcasted_iota(jnp.int32, (TILE, NUM_CODES), 1)
              == idx[:, None]).astype(jnp.float32)
    zq = jnp.dot(onehot, cb_ref[...].T, preferred_element_type=jnp.float32)

    diff = zq - z
    loss_ref[...] = jnp.sum(diff * diff).reshape(1, 1, 1)

    hd = jnp.maximum(
        jnp.dot(zq, W3_ref[...], preferred_element_type=jnp.float32) + b3_ref[...], 0.0)
    out_ref[...] = jnp.dot(hd, W4_ref[...], preferred_element_type=jnp.float32) + b4_ref[...]


@jax.jit
def kernel(x, W1, b1, W2, b2, codebook, W3, b3, W4, b4):
    grid = N // TILE
    cb_t = codebook.T
    w2sum = jnp.sum(codebook * codebook, axis=1)[None, :]

    full = lambda shape: pl.BlockSpec(shape, lambda i: (0,) * len(shape))
    x_hat, loss_parts = pl.pallas_call(
        _fused_body,
        grid=(grid,),
        in_specs=[
            pl.BlockSpec((TILE, D_IN), lambda i: (i, 0)),
            full((D_IN, H1)),
            full((1, H1)),
            full((H1, H2)),
            full((1, H2)),
            full((CODE_DIM, NUM_CODES)),
            full((1, NUM_CODES)),
            full((H2, H1)),
            full((1, H1)),
            full((H1, D_IN)),
            full((1, D_IN)),
        ],
        out_specs=[
            pl.BlockSpec((TILE, D_IN), lambda i: (i, 0)),
            pl.BlockSpec((1, 1, 1), lambda i: (i, 0, 0)),
        ],
        out_shape=[
            jax.ShapeDtypeStruct((N, D_IN), jnp.float32),
            jax.ShapeDtypeStruct((grid, 1, 1), jnp.float32),
        ],
        compiler_params=pltpu.CompilerParams(
            dimension_semantics=("parallel",),
        ),
    )(x, W1, b1[None, :], W2, b2[None, :], cb_t, w2sum,
      W3, b3[None, :], W4, b4[None, :])

    vq_loss = jnp.sum(loss_parts) * ((1.0 + COMMITMENT_COST) / (N * H2))
    return (x_hat, vq_loss)


# speedup vs baseline: 1.1063x; 1.1063x over previous
import functools

import jax
import jax.numpy as jnp
from jax.experimental import pallas as pl
from jax.experimental.pallas import tpu as pltpu

N, D_IN = 16384, 768
H1, H2 = 1024, 256
NUM_CODES, CODE_DIM = 256, 256
COMMITMENT_COST = 0.25

TILE = 1024


def _fused_body(x_ref, W1_ref, b1_ref, W2_ref, b2_ref, cb_ref, w2sum_ref,
                W3_ref, b3_ref, W4_ref, b4_ref, out_ref, loss_ref):
    x = x_ref[...]
    h = jnp.maximum(
        jnp.dot(x, W1_ref[...], preferred_element_type=jnp.float32) + b1_ref[...], 0.0)
    z = jnp.maximum(
        jnp.dot(h, W2_ref[...], preferred_element_type=jnp.float32) + b2_ref[...], 0.0)

    zc = jnp.dot(z, cb_ref[...], preferred_element_type=jnp.float32)
    z2 = jnp.sum(z * z, axis=1, keepdims=True)
    d2 = jnp.maximum(z2 + w2sum_ref[...] - 2.0 * zc, 0.0)
    idx = jnp.argmin(d2, axis=1)

    onehot = (jax.lax.broadcasted_iota(jnp.int32, (TILE, NUM_CODES), 1)
              == idx[:, None]).astype(jnp.float32)
    zq = jnp.dot(onehot, cb_ref[...].T, preferred_element_type=jnp.float32)

    diff = zq - z
    loss_ref[...] = jnp.sum(diff * diff).reshape(1, 1, 1)

    hd = jnp.maximum(
        jnp.dot(zq, W3_ref[...], preferred_element_type=jnp.float32) + b3_ref[...], 0.0)
    out_ref[...] = jnp.dot(hd, W4_ref[...], preferred_element_type=jnp.float32) + b4_ref[...]


@jax.jit
def kernel(x, W1, b1, W2, b2, codebook, W3, b3, W4, b4):
    grid = N // TILE
    cb_t = codebook.T
    w2sum = jnp.sum(codebook * codebook, axis=1)[None, :]

    full = lambda shape: pl.BlockSpec(shape, lambda i: (0,) * len(shape))
    x_hat, loss_parts = pl.pallas_call(
        _fused_body,
        grid=(grid,),
        in_specs=[
            pl.BlockSpec((TILE, D_IN), lambda i: (i, 0)),
            full((D_IN, H1)),
            full((1, H1)),
            full((H1, H2)),
            full((1, H2)),
            full((CODE_DIM, NUM_CODES)),
            full((1, NUM_CODES)),
            full((H2, H1)),
            full((1, H1)),
            full((H1, D_IN)),
            full((1, D_IN)),
        ],
        out_specs=[
            pl.BlockSpec((TILE, D_IN), lambda i: (i, 0)),
            pl.BlockSpec((1, 1, 1), lambda i: (i, 0, 0)),
        ],
        out_shape=[
            jax.ShapeDtypeStruct((N, D_IN), jnp.float32),
            jax.ShapeDtypeStruct((grid, 1, 1), jnp.float32),
        ],
        compiler_params=pltpu.CompilerParams(
            dimension_semantics=("parallel",),
        ),
    )(x, W1, b1[None, :], W2, b2[None, :], cb_t, w2sum,
      W3, b3[None, :], W4, b4[None, :])

    vq_loss = jnp.sum(loss_parts) * ((1.0 + COMMITMENT_COST) / (N * H2))
    return (x_hat, vq_loss)


# speedup vs baseline: 1.2581x; 1.1373x over previous
import functools

import jax
import jax.numpy as jnp
from jax.experimental import pallas as pl
from jax.experimental.pallas import tpu as pltpu

N, D_IN = 16384, 768
H1, H2 = 1024, 256
NUM_CODES, CODE_DIM = 256, 256
COMMITMENT_COST = 0.25

TILE = 2048


def _fused_body(x_ref, W1_ref, b1_ref, W2_ref, b2_ref, cb_ref, w2sum_ref,
                W3_ref, b3_ref, W4_ref, b4_ref, out_ref, loss_ref):
    x = x_ref[...]
    h = jnp.maximum(
        jnp.dot(x, W1_ref[...], preferred_element_type=jnp.float32) + b1_ref[...], 0.0)
    z = jnp.maximum(
        jnp.dot(h, W2_ref[...], preferred_element_type=jnp.float32) + b2_ref[...], 0.0)

    zc = jnp.dot(z, cb_ref[...], preferred_element_type=jnp.float32)
    z2 = jnp.sum(z * z, axis=1, keepdims=True)
    d2 = jnp.maximum(z2 + w2sum_ref[...] - 2.0 * zc, 0.0)
    idx = jnp.argmin(d2, axis=1)

    onehot = (jax.lax.broadcasted_iota(jnp.int32, (TILE, NUM_CODES), 1)
              == idx[:, None]).astype(jnp.float32)
    zq = jnp.dot(onehot, cb_ref[...].T, preferred_element_type=jnp.float32)

    diff = zq - z
    loss_ref[...] = jnp.sum(diff * diff).reshape(1, 1, 1)

    hd = jnp.maximum(
        jnp.dot(zq, W3_ref[...], preferred_element_type=jnp.float32) + b3_ref[...], 0.0)
    out_ref[...] = jnp.dot(hd, W4_ref[...], preferred_element_type=jnp.float32) + b4_ref[...]


@jax.jit
def kernel(x, W1, b1, W2, b2, codebook, W3, b3, W4, b4):
    grid = N // TILE
    cb_t = codebook.T
    w2sum = jnp.sum(codebook * codebook, axis=1)[None, :]

    full = lambda shape: pl.BlockSpec(shape, lambda i: (0,) * len(shape))
    x_hat, loss_parts = pl.pallas_call(
        _fused_body,
        grid=(grid,),
        in_specs=[
            pl.BlockSpec((TILE, D_IN), lambda i: (i, 0)),
            full((D_IN, H1)),
            full((1, H1)),
            full((H1, H2)),
            full((1, H2)),
            full((CODE_DIM, NUM_CODES)),
            full((1, NUM_CODES)),
            full((H2, H1)),
            full((1, H1)),
            full((H1, D_IN)),
            full((1, D_IN)),
        ],
        out_specs=[
            pl.BlockSpec((TILE, D_IN), lambda i: (i, 0)),
            pl.BlockSpec((1, 1, 1), lambda i: (i, 0, 0)),
        ],
        out_shape=[
            jax.ShapeDtypeStruct((N, D_IN), jnp.float32),
            jax.ShapeDtypeStruct((grid, 1, 1), jnp.float32),
        ],
        compiler_params=pltpu.CompilerParams(
            dimension_semantics=("parallel",),
        ),
    )(x, W1, b1[None, :], W2, b2[None, :], cb_t, w2sum,
      W3, b3[None, :], W4, b4[None, :])

    vq_loss = jnp.sum(loss_parts) * ((1.0 + COMMITMENT_COST) / (N * H2))
    return (x_hat, vq_loss)
